# baseline (speedup 1.0000x reference)
import functools

import jax
import jax.numpy as jnp
from jax import lax
from jax.experimental import pallas as pl
from jax.experimental.pallas import tpu as pltpu
from jax.experimental.pallas import tpu_sc as plsc

N = 10000
E = 160000
D = 128
DE = 16
DU = 16
B = 64
NSLOT = 8
NCONV = 4

NW = 32
CH = 128
CPW = 40
EPAD = NW * CH * CPW
NTRASH = 8
NP8 = N + NTRASH

ROWS = 1000
NT = N // ROWS
EROWS = 2048
ET = EPAD // EROWS
RZA = 624
RZB = N - 15 * RZA

_f32 = jnp.float32


def _dot(a, b):
    return jnp.dot(a, b, preferred_element_type=_f32)


def _dot_exact(a, b):
    return jnp.dot(a, b, preferred_element_type=_f32,
                   precision=lax.Precision.HIGHEST)


def _pad_w3(w3):
    return jnp.pad(w3, ((0, 0), (0, 127)))


def _lin_body(x_ref, w_ref, b_ref, o_ref):
    o_ref[...] = _dot(x_ref[...], w_ref[...]) + b_ref[...]


def _node_lin(x, w, b):
    return pl.pallas_call(
        _lin_body,
        grid=(NT,),
        in_specs=[pl.BlockSpec((ROWS, D), lambda i: (i, 0)),
                  pl.BlockSpec((D, D), lambda i: (0, 0)),
                  pl.BlockSpec((1, D), lambda i: (0, 0))],
        out_specs=pl.BlockSpec((ROWS, D), lambda i: (i, 0)),
        out_shape=jax.ShapeDtypeStruct((N, D), _f32),
        interpret=False,
    )(x, w, b)


def _pre_body(x_ref, b2_ref, wsrc_ref, wdst_ref, u_ref, wub_ref, bm_ref,
              xs_ref, xd2_ref):
    x = x_ref[...]
    xs_ref[...] = _dot(x, wsrc_ref[...])
    ub = _dot(u_ref[...], wub_ref[...])
    bo = (b2_ref[...] == lax.broadcasted_iota(jnp.int32, (1, B), 1)
          ).astype(_f32)
    xd2_ref[...] = (_dot(x, wdst_ref[...]) + _dot_exact(bo, ub)
                    + bm_ref[...])


def _pre_node(x, batch2, wsrc, wdst, u, wub, bm):
    return pl.pallas_call(
        _pre_body,
        grid=(NT,),
        in_specs=[pl.BlockSpec((ROWS, D), lambda i: (i, 0)),
                  pl.BlockSpec((ROWS, 1), lambda i: (i, 0)),
                  pl.BlockSpec((D, D), lambda i: (0, 0)),
                  pl.BlockSpec((D, D), lambda i: (0, 0)),
                  pl.BlockSpec((B, DU), lambda i: (0, 0)),
                  pl.BlockSpec((DU, D), lambda i: (0, 0)),
                  pl.BlockSpec((1, D), lambda i: (0, 0))],
        out_specs=[pl.BlockSpec((ROWS, D), lambda i: (i, 0)),
                   pl.BlockSpec((ROWS, D), lambda i: (i, 0))],
        out_shape=[jax.ShapeDtypeStruct((N, D), _f32),
                   jax.ShapeDtypeStruct((NP8, D), _f32)],
        interpret=False,
    )(x, batch2, wsrc, wdst, u, wub, bm)


def _eb2_body(ea_ref, es_ref, wme_ref, cs_ref, o_ref):
    so = (es_ref[...] == lax.broadcasted_iota(jnp.int32, (1, NSLOT), 1)
          ).astype(_f32)
    o_ref[...] = (_dot(ea_ref[...], wme_ref[...])
                  + _dot_exact(so, cs_ref[...]))


def _edge_pre(eap, esp, wme, cslot):
    return pl.pallas_call(
        _eb2_body,
        grid=(ET,),
        in_specs=[pl.BlockSpec((EROWS, DE), lambda i: (i, 0)),
                  pl.BlockSpec((EROWS, 1), lambda i: (i, 0)),
                  pl.BlockSpec((DE, D), lambda i: (0, 0)),
                  pl.BlockSpec((NSLOT, D), lambda i: (0, 0))],
        out_specs=pl.BlockSpec((EROWS, D), lambda i: (i, 0)),
        out_shape=jax.ShapeDtypeStruct((EPAD, D), _f32),
        interpret=False,
    )(eap, esp, wme, cslot)


def _upd_body(residual, x_ref, a0_ref, a1_ref, res_ref, w1_ref, w2_ref,
              bu_ref, o_ref):
    agg = a0_ref[...] + a1_ref[...]
    o = _dot(x_ref[...], w1_ref[...]) + _dot(agg, w2_ref[...]) + bu_ref[...]
    if residual:
        o = o + res_ref[...]
    o_ref[...] = jnp.maximum(o, 0.0)


def _update(x, a0, a1, res, w1, w2, bu, residual):
    return pl.pallas_call(
        functools.partial(_upd_body, residual),
        grid=(NT,),
        in_specs=[pl.BlockSpec((ROWS, D), lambda i: (i, 0)),
                  pl.BlockSpec((ROWS, D), lambda i: (i, 0)),
                  pl.BlockSpec((ROWS, D), lambda i: (i, 0)),
                  pl.BlockSpec((ROWS, D), lambda i: (i, 0)),
                  pl.BlockSpec((D, D), lambda i: (0, 0)),
                  pl.BlockSpec((D, D), lambda i: (0, 0)),
                  pl.BlockSpec((1, D), lambda i: (0, 0))],
        out_specs=pl.BlockSpec((ROWS, D), lambda i: (i, 0)),
        out_shape=jax.ShapeDtypeStruct((N, D), _f32),
        interpret=False,
    )(x, a0, a1, res, w1, w2, bu)


def _head_body(x_ref, gw1, gb1, gw2, gb2, gw3, nw1, nb1, nw2, nb2, nw3,
               g_ref, h_ref):
    x = x_ref[...]
    t = jnp.maximum(_dot(x, gw1[...]) + gb1[...], 0.0)
    t = jnp.maximum(_dot(t, gw2[...]) + gb2[...], 0.0)
    g_ref[...] = _dot(t, gw3[...])
    t = jnp.maximum(_dot(x, nw1[...]) + nb1[...], 0.0)
    t = jnp.maximum(_dot(t, nw2[...]) + nb2[...], 0.0)
    h_ref[...] = _dot(t, nw3[...])


def _heads(x, gw1, gb1, gw2, gb2, gw3, nw1, nb1, nw2, nb2, nw3):
    wb = lambda r, c: pl.BlockSpec((r, c), lambda i: (0, 0))
    return pl.pallas_call(
        _head_body,
        grid=(NT,),
        in_specs=[pl.BlockSpec((ROWS, D), lambda i: (i, 0)),
                  wb(D, 128), wb(1, 128), wb(128, 128), wb(1, 128),
                  wb(128, 128),
                  wb(D, 128), wb(1, 128), wb(128, 128), wb(1, 128),
                  wb(128, 128)],
        out_specs=[pl.BlockSpec((ROWS, D), lambda i: (i, 0)),
                   pl.BlockSpec((ROWS, D), lambda i: (i, 0))],
        out_shape=[jax.ShapeDtypeStruct((N, D), _f32),
                   jax.ShapeDtypeStruct((N, D), _f32)],
        interpret=False,
    )(x, gw1, gb1, gw2, gb2, gw3, nw1, nb1, nw2, nb2, nw3)


def _pool_body(g_ref, h_ref, b2_ref, gb3_ref, nb3_ref, o_ref,
               gmax_s, num_s, den_s):
    p = pl.program_id(0)
    t = pl.program_id(1)
    bo = b2_ref[...] == lax.broadcasted_iota(jnp.int32, (1, B), 1)

    @pl.when((p == 0) & (t == 0))
    def _():
        gmax_s[...] = jnp.full((1, B), -1e30, _f32)
        num_s[...] = jnp.zeros((1, B), _f32)
        den_s[...] = jnp.zeros((1, B), _f32)

    g = g_ref[...][:, 0:1]

    @pl.when(p == 0)
    def _():
        pm = jnp.max(jnp.where(bo, g, -1e30), axis=0, keepdims=True)
        gmax_s[...] = jnp.maximum(gmax_s[...], pm)

    @pl.when(p == 1)
    def _():
        bof = bo.astype(_f32)
        gm = jnp.sum(bof * gmax_s[...], axis=1, keepdims=True)
        ge = jnp.exp(g - gm)
        den_p = jnp.sum(bof * ge, axis=0, keepdims=True)
        num_p = jnp.sum(bof * (ge * h_ref[...][:, 0:1]), axis=0,
                        keepdims=True)
        num_s[...] = num_s[...] + num_p
        den_s[...] = den_s[...] + den_p

    @pl.when((p == 1) & (t == NT - 1))
    def _():
        den = den_s[...]
        o_ref[...] = jnp.where(den > 0.5,
                               num_s[...] / jnp.maximum(den, 1e-20)
                               + nb3_ref[...] + gb3_ref[...] * 0.0,
                               0.0)


def _pool(g, h, batch2, gb3, nb3):
    return pl.pallas_call(
        _pool_body,
        grid=(2, NT),
        in_specs=[pl.BlockSpec((ROWS, D), lambda p, t: (t, 0)),
                  pl.BlockSpec((ROWS, D), lambda p, t: (t, 0)),
                  pl.BlockSpec((ROWS, 1), lambda p, t: (t, 0)),
                  pl.BlockSpec((1, 1), lambda p, t: (0, 0)),
                  pl.BlockSpec((1, 1), lambda p, t: (0, 0))],
        out_specs=pl.BlockSpec((1, B), lambda p, t: (0, 0)),
        out_shape=jax.ShapeDtypeStruct((1, B), _f32),
        scratch_shapes=[pltpu.VMEM((1, B), _f32),
                        pltpu.VMEM((1, B), _f32),
                        pltpu.VMEM((1, B), _f32)],
        interpret=False,
    )(g, h, batch2, gb3, nb3)


def _sc_body(xs_hbm, xd2_hbm, eb2_hbm, src_hbm, dst_hbm, zeros_hbm, out_hbm,
             srcv, dstv, bufa, bufb, bufe, shared, sema, semb, seme):
    c = lax.axis_index("c")
    s = lax.axis_index("s")
    wid = s * 2 + c

    @pl.when(s < 15)
    def _():
        o = pl.multiple_of(s * RZA, 8)
        pltpu.sync_copy(zeros_hbm.at[pl.ds(o, RZA)],
                        shared.at[pl.ds(o, RZA)])

    @pl.when(s == 15)
    def _():
        pltpu.sync_copy(zeros_hbm.at[pl.ds(15 * RZA, RZB)],
                        shared.at[pl.ds(15 * RZA, RZB)])

    plsc.subcore_barrier()

    ebase = wid * (CPW * CH)

    def chunk(k, carry):
        off = pl.multiple_of(ebase + k * CH, CH)
        pltpu.sync_copy(src_hbm.at[pl.ds(off, CH)], srcv)
        pltpu.sync_copy(dst_hbm.at[pl.ds(off, CH)], dstv)
        cpa = pltpu.async_copy(xs_hbm.at[srcv], bufa, sema)
        cpb = pltpu.async_copy(xd2_hbm.at[dstv], bufb, semb)
        cpe = pltpu.async_copy(eb2_hbm.at[pl.ds(off, CH)], bufe, seme)
        cpa.wait()
        cpb.wait()
        cpe.wait()

        def inner(e, carry2):
            for d2 in range(D // 16):
                sl = pl.ds(d2 * 16, 16)
                v = bufa[e, sl] + bufb[e, sl] + bufe[e, sl]
                bufe[e, sl] = jnp.maximum(v, 0.0)
            return carry2

        lax.fori_loop(0, CH, inner, 0)
        pltpu.sync_copy(bufe, shared.at[dstv], add=True)
        return carry

    lax.fori_loop(0, CPW, chunk, 0)
    plsc.subcore_barrier()

    @pl.when(s < 15)
    def _():
        o = pl.multiple_of(s * RZA, 8)
        pltpu.sync_copy(shared.at[pl.ds(o, RZA)],
                        out_hbm.at[c, pl.ds(o, RZA)])

    @pl.when(s == 15)
    def _():
        pltpu.sync_copy(shared.at[pl.ds(15 * RZA, RZB)],
                        out_hbm.at[c, pl.ds(15 * RZA, RZB)])


@functools.cache
def _sc_edges_fn():
    return pl.kernel(
        _sc_body,
        out_type=jax.ShapeDtypeStruct((2, N, D), _f32),
        mesh=plsc.VectorSubcoreMesh(core_axis_name="c",
                                    subcore_axis_name="s"),
        scratch_types=[pltpu.VMEM((CH,), jnp.int32),
                       pltpu.VMEM((CH,), jnp.int32),
                       pltpu.VMEM((CH, D), _f32),
                       pltpu.VMEM((CH, D), _f32),
                       pltpu.VMEM((CH, D), _f32),
                       pltpu.VMEM_SHARED((NP8, D), _f32),
                       pltpu.SemaphoreType.DMA,
                       pltpu.SemaphoreType.DMA,
                       pltpu.SemaphoreType.DMA],
    )


def _sc_edges(*args):
    return _sc_edges_fn()(*args)


def kernel(node_attr, edge_index, edge_slot, edge_attr, u, batch,
           node_lin_W, node_lin_b, conv_Wm, conv_bm, conv_slot, conv_Wu,
           conv_bu, gate_W1, gate_b1, gate_W2, gate_b2, gate_W3, gate_b3,
           nn_W1, nn_b1, nn_W2, nn_b2, nn_W3, nn_b3):
    src = edge_index[0]
    dst = edge_index[1]
    pad = EPAD - E
    srcp = jnp.concatenate([src, jnp.zeros((pad,), jnp.int32)])
    dstp = jnp.concatenate([dst, jnp.full((pad,), N, jnp.int32)])
    eap = jnp.concatenate([edge_attr, jnp.zeros((pad, DE), _f32)])
    esp = jnp.concatenate([edge_slot, jnp.zeros((pad,), jnp.int32)]
                          ).reshape(EPAD, 1)
    batch2 = batch.reshape(N, 1)
    zeros = jnp.zeros((N, D), _f32)

    x = _node_lin(node_attr, node_lin_W, node_lin_b.reshape(1, D))

    def conv(i, xin, res, residual):
        wm = conv_Wm[i]
        eb2 = _edge_pre(eap, esp, wm[2 * D:2 * D + DE], conv_slot[i])
        xs, xd2p = _pre_node(xin, batch2, wm[:D], wm[D:2 * D], u,
                             wm[2 * D + DE:], conv_bm[i].reshape(1, D))
        agg2 = _sc_edges(xs, xd2p, eb2, srcp, dstp, zeros)
        wu = conv_Wu[i]
        return _update(xin, agg2[0], agg2[1], res, wu[:D], wu[D:],
                       conv_bu[i].reshape(1, D), residual)

    ci = 0
    for _ in range(2):
        h = conv(ci, x, x, False)
        x = conv(ci + 1, h, x, True)
        ci += 2

    g, hv = _heads(x, gate_W1, gate_b1.reshape(1, 128), gate_W2,
                   gate_b2.reshape(1, 128), _pad_w3(gate_W3),
                   nn_W1, nn_b1.reshape(1, 128), nn_W2,
                   nn_b2.reshape(1, 128), _pad_w3(nn_W3))
    out = _pool(g, hv, batch2, gate_b3.reshape(1, 1), nn_b3.reshape(1, 1))
    return out.reshape(B, 1)


# speedup vs baseline: 1.1883x; 1.1883x over previous
import functools

import jax
import jax.numpy as jnp
from jax import lax
from jax.experimental import pallas as pl
from jax.experimental.pallas import tpu as pltpu
from jax.experimental.pallas import tpu_sc as plsc

N = 10000
E = 160000
D = 128
DE = 16
DU = 16
B = 64
NSLOT = 8
NCONV = 4

NW = 32
CH = 64
CPW = 80
EPAD = NW * CH * CPW
NTRASH = 8
NP8 = N + NTRASH

ROWS = 1000
NT = N // ROWS
EROWS = 2048
ET = EPAD // EROWS
RZA = 624
RZB = N - 15 * RZA

_f32 = jnp.float32


def _dot(a, b):
    return jnp.dot(a, b, preferred_element_type=_f32)


def _dot_exact(a, b):
    return jnp.dot(a, b, preferred_element_type=_f32,
                   precision=lax.Precision.HIGHEST)


def _pad_w3(w3):
    return jnp.pad(w3, ((0, 0), (0, 127)))


def _lin_body(x_ref, w_ref, b_ref, o_ref):
    o_ref[...] = _dot(x_ref[...], w_ref[...]) + b_ref[...]


def _node_lin(x, w, b):
    return pl.pallas_call(
        _lin_body,
        grid=(NT,),
        in_specs=[pl.BlockSpec((ROWS, D), lambda i: (i, 0)),
                  pl.BlockSpec((D, D), lambda i: (0, 0)),
                  pl.BlockSpec((1, D), lambda i: (0, 0))],
        out_specs=pl.BlockSpec((ROWS, D), lambda i: (i, 0)),
        out_shape=jax.ShapeDtypeStruct((N, D), _f32),
        interpret=False,
    )(x, w, b)


def _pre_body(x_ref, b2_ref, wsrc_ref, wdst_ref, u_ref, wub_ref, bm_ref,
              xs_ref, xd2_ref):
    x = x_ref[...]
    xs_ref[...] = _dot(x, wsrc_ref[...])
    ub = _dot(u_ref[...], wub_ref[...])
    bo = (b2_ref[...] == lax.broadcasted_iota(jnp.int32, (1, B), 1)
          ).astype(_f32)
    xd2_ref[...] = (_dot(x, wdst_ref[...]) + _dot_exact(bo, ub)
                    + bm_ref[...])


def _pre_node(x, batch2, wsrc, wdst, u, wub, bm):
    return pl.pallas_call(
        _pre_body,
        grid=(NT,),
        in_specs=[pl.BlockSpec((ROWS, D), lambda i: (i, 0)),
                  pl.BlockSpec((ROWS, 1), lambda i: (i, 0)),
                  pl.BlockSpec((D, D), lambda i: (0, 0)),
                  pl.BlockSpec((D, D), lambda i: (0, 0)),
                  pl.BlockSpec((B, DU), lambda i: (0, 0)),
                  pl.BlockSpec((DU, D), lambda i: (0, 0)),
                  pl.BlockSpec((1, D), lambda i: (0, 0))],
        out_specs=[pl.BlockSpec((ROWS, D), lambda i: (i, 0)),
                   pl.BlockSpec((ROWS, D), lambda i: (i, 0))],
        out_shape=[jax.ShapeDtypeStruct((N, D), _f32),
                   jax.ShapeDtypeStruct((NP8, D), _f32)],
        interpret=False,
    )(x, batch2, wsrc, wdst, u, wub, bm)


def _eb2_body(ea_ref, es_ref, wme_ref, cs_ref, o_ref):
    so = (es_ref[...] == lax.broadcasted_iota(jnp.int32, (1, NSLOT), 1)
          ).astype(_f32)
    o_ref[...] = (_dot(ea_ref[...], wme_ref[...])
                  + _dot_exact(so, cs_ref[...]))


def _edge_pre(eap, esp, wme, cslot):
    return pl.pallas_call(
        _eb2_body,
        grid=(ET,),
        in_specs=[pl.BlockSpec((EROWS, DE), lambda i: (i, 0)),
                  pl.BlockSpec((EROWS, 1), lambda i: (i, 0)),
                  pl.BlockSpec((DE, D), lambda i: (0, 0)),
                  pl.BlockSpec((NSLOT, D), lambda i: (0, 0))],
        out_specs=pl.BlockSpec((EROWS, D), lambda i: (i, 0)),
        out_shape=jax.ShapeDtypeStruct((EPAD, D), _f32),
        interpret=False,
    )(eap, esp, wme, cslot)


def _upd_body(residual, x_ref, a0_ref, a1_ref, res_ref, w1_ref, w2_ref,
              bu_ref, o_ref):
    agg = a0_ref[...] + a1_ref[...]
    o = _dot(x_ref[...], w1_ref[...]) + _dot(agg, w2_ref[...]) + bu_ref[...]
    if residual:
        o = o + res_ref[...]
    o_ref[...] = jnp.maximum(o, 0.0)


def _update(x, a0, a1, res, w1, w2, bu, residual):
    return pl.pallas_call(
        functools.partial(_upd_body, residual),
        grid=(NT,),
        in_specs=[pl.BlockSpec((ROWS, D), lambda i: (i, 0)),
                  pl.BlockSpec((ROWS, D), lambda i: (i, 0)),
                  pl.BlockSpec((ROWS, D), lambda i: (i, 0)),
                  pl.BlockSpec((ROWS, D), lambda i: (i, 0)),
                  pl.BlockSpec((D, D), lambda i: (0, 0)),
                  pl.BlockSpec((D, D), lambda i: (0, 0)),
                  pl.BlockSpec((1, D), lambda i: (0, 0))],
        out_specs=pl.BlockSpec((ROWS, D), lambda i: (i, 0)),
        out_shape=jax.ShapeDtypeStruct((N, D), _f32),
        interpret=False,
    )(x, a0, a1, res, w1, w2, bu)


def _head_body(x_ref, gw1, gb1, gw2, gb2, gw3, nw1, nb1, nw2, nb2, nw3,
               g_ref, h_ref):
    x = x_ref[...]
    t = jnp.maximum(_dot(x, gw1[...]) + gb1[...], 0.0)
    t = jnp.maximum(_dot(t, gw2[...]) + gb2[...], 0.0)
    g_ref[...] = _dot(t, gw3[...])
    t = jnp.maximum(_dot(x, nw1[...]) + nb1[...], 0.0)
    t = jnp.maximum(_dot(t, nw2[...]) + nb2[...], 0.0)
    h_ref[...] = _dot(t, nw3[...])


def _heads(x, gw1, gb1, gw2, gb2, gw3, nw1, nb1, nw2, nb2, nw3):
    wb = lambda r, c: pl.BlockSpec((r, c), lambda i: (0, 0))
    return pl.pallas_call(
        _head_body,
        grid=(NT,),
        in_specs=[pl.BlockSpec((ROWS, D), lambda i: (i, 0)),
                  wb(D, 128), wb(1, 128), wb(128, 128), wb(1, 128),
                  wb(128, 128),
                  wb(D, 128), wb(1, 128), wb(128, 128), wb(1, 128),
                  wb(128, 128)],
        out_specs=[pl.BlockSpec((ROWS, D), lambda i: (i, 0)),
                   pl.BlockSpec((ROWS, D), lambda i: (i, 0))],
        out_shape=[jax.ShapeDtypeStruct((N, D), _f32),
                   jax.ShapeDtypeStruct((N, D), _f32)],
        interpret=False,
    )(x, gw1, gb1, gw2, gb2, gw3, nw1, nb1, nw2, nb2, nw3)


def _pool_body(g_ref, h_ref, b2_ref, gb3_ref, nb3_ref, o_ref,
               gmax_s, num_s, den_s):
    p = pl.program_id(0)
    t = pl.program_id(1)
    bo = b2_ref[...] == lax.broadcasted_iota(jnp.int32, (1, B), 1)

    @pl.when((p == 0) & (t == 0))
    def _():
        gmax_s[...] = jnp.full((1, B), -1e30, _f32)
        num_s[...] = jnp.zeros((1, B), _f32)
        den_s[...] = jnp.zeros((1, B), _f32)

    g = g_ref[...][:, 0:1]

    @pl.when(p == 0)
    def _():
        pm = jnp.max(jnp.where(bo, g, -1e30), axis=0, keepdims=True)
        gmax_s[...] = jnp.maximum(gmax_s[...], pm)

    @pl.when(p == 1)
    def _():
        bof = bo.astype(_f32)
        gm = jnp.sum(bof * gmax_s[...], axis=1, keepdims=True)
        ge = jnp.exp(g - gm)
        den_p = jnp.sum(bof * ge, axis=0, keepdims=True)
        num_p = jnp.sum(bof * (ge * h_ref[...][:, 0:1]), axis=0,
                        keepdims=True)
        num_s[...] = num_s[...] + num_p
        den_s[...] = den_s[...] + den_p

    @pl.when((p == 1) & (t == NT - 1))
    def _():
        den = den_s[...]
        o_ref[...] = jnp.where(den > 0.5,
                               num_s[...] / jnp.maximum(den, 1e-20)
                               + nb3_ref[...] + gb3_ref[...] * 0.0,
                               0.0)


def _pool(g, h, batch2, gb3, nb3):
    return pl.pallas_call(
        _pool_body,
        grid=(2, NT),
        in_specs=[pl.BlockSpec((ROWS, D), lambda p, t: (t, 0)),
                  pl.BlockSpec((ROWS, D), lambda p, t: (t, 0)),
                  pl.BlockSpec((ROWS, 1), lambda p, t: (t, 0)),
                  pl.BlockSpec((1, 1), lambda p, t: (0, 0)),
                  pl.BlockSpec((1, 1), lambda p, t: (0, 0))],
        out_specs=pl.BlockSpec((1, B), lambda p, t: (0, 0)),
        out_shape=jax.ShapeDtypeStruct((1, B), _f32),
        scratch_shapes=[pltpu.VMEM((1, B), _f32),
                        pltpu.VMEM((1, B), _f32),
                        pltpu.VMEM((1, B), _f32)],
        interpret=False,
    )(g, h, batch2, gb3, nb3)


def _sc_body(xs_hbm, xd2_hbm, eb2_hbm, src_hbm, dst_hbm, zeros_hbm, out_hbm,
             srcv0, srcv1, dstv0, dstv1, bufa0, bufa1, bufb0, bufb1,
             bufe0, bufe1, shared, sema0, sema1, semb0, semb1, seme0,
             seme1):
    srcv = (srcv0, srcv1)
    dstv = (dstv0, dstv1)
    bufa = (bufa0, bufa1)
    bufb = (bufb0, bufb1)
    bufe = (bufe0, bufe1)
    sema = (sema0, sema1)
    semb = (semb0, semb1)
    seme = (seme0, seme1)
    c = lax.axis_index("c")
    s = lax.axis_index("s")
    wid = s * 2 + c

    @pl.when(s < 15)
    def _():
        o = pl.multiple_of(s * RZA, 8)
        pltpu.sync_copy(zeros_hbm.at[pl.ds(o, RZA)],
                        shared.at[pl.ds(o, RZA)])

    @pl.when(s == 15)
    def _():
        pltpu.sync_copy(zeros_hbm.at[pl.ds(15 * RZA, RZB)],
                        shared.at[pl.ds(15 * RZA, RZB)])

    plsc.subcore_barrier()

    ebase = wid * (CPW * CH)

    def issue(k, b):
        off = pl.multiple_of(ebase + k * CH, CH)
        pltpu.sync_copy(src_hbm.at[pl.ds(off, CH)], srcv[b])
        pltpu.sync_copy(dst_hbm.at[pl.ds(off, CH)], dstv[b])
        pltpu.async_copy(xs_hbm.at[srcv[b]], bufa[b], sema[b])
        pltpu.async_copy(xd2_hbm.at[dstv[b]], bufb[b], semb[b])
        pltpu.async_copy(eb2_hbm.at[pl.ds(off, CH)], bufe[b], seme[b])

    def finish(b):
        pltpu.make_async_copy(eb2_hbm.at[pl.ds(0, CH)], bufa[b],
                              sema[b]).wait()
        pltpu.make_async_copy(eb2_hbm.at[pl.ds(0, CH)], bufb[b],
                              semb[b]).wait()
        pltpu.make_async_copy(eb2_hbm.at[pl.ds(0, CH)], bufe[b],
                              seme[b]).wait()

        def inner(e, carry2):
            for d2 in range(D // 16):
                sl = pl.ds(d2 * 16, 16)
                v = bufa[b][e, sl] + bufb[b][e, sl] + bufe[b][e, sl]
                bufe[b][e, sl] = jnp.maximum(v, 0.0)
            return carry2

        lax.fori_loop(0, CH, inner, 0)
        pltpu.sync_copy(bufe[b], shared.at[dstv[b]], add=True)

    issue(0, 0)

    def pair(i, carry):
        k0 = i * 2
        issue(k0 + 1, 1)
        finish(0)

        @pl.when(k0 + 2 < CPW)
        def _():
            issue(k0 + 2, 0)

        finish(1)
        return carry

    lax.fori_loop(0, CPW // 2, pair, 0)
    plsc.subcore_barrier()

    @pl.when(s < 15)
    def _():
        o = pl.multiple_of(s * RZA, 8)
        pltpu.sync_copy(shared.at[pl.ds(o, RZA)],
                        out_hbm.at[c, pl.ds(o, RZA)])

    @pl.when(s == 15)
    def _():
        pltpu.sync_copy(shared.at[pl.ds(15 * RZA, RZB)],
                        out_hbm.at[c, pl.ds(15 * RZA, RZB)])


@functools.cache
def _sc_edges_fn():
    return pl.kernel(
        _sc_body,
        out_type=jax.ShapeDtypeStruct((2, N, D), _f32),
        mesh=plsc.VectorSubcoreMesh(core_axis_name="c",
                                    subcore_axis_name="s"),
        scratch_types=([pltpu.VMEM((CH,), jnp.int32)] * 4
                       + [pltpu.VMEM((CH, D), _f32)] * 6
                       + [pltpu.VMEM_SHARED((NP8, D), _f32)]
                       + [pltpu.SemaphoreType.DMA] * 6),
    )


def _sc_edges(*args):
    return _sc_edges_fn()(*args)


def kernel(node_attr, edge_index, edge_slot, edge_attr, u, batch,
           node_lin_W, node_lin_b, conv_Wm, conv_bm, conv_slot, conv_Wu,
           conv_bu, gate_W1, gate_b1, gate_W2, gate_b2, gate_W3, gate_b3,
           nn_W1, nn_b1, nn_W2, nn_b2, nn_W3, nn_b3):
    src = edge_index[0]
    dst = edge_index[1]
    pad = EPAD - E
    srcp = jnp.concatenate([src, jnp.zeros((pad,), jnp.int32)])
    dstp = jnp.concatenate([dst, jnp.full((pad,), N, jnp.int32)])
    eap = jnp.concatenate([edge_attr, jnp.zeros((pad, DE), _f32)])
    esp = jnp.concatenate([edge_slot, jnp.zeros((pad,), jnp.int32)]
                          ).reshape(EPAD, 1)
    batch2 = batch.reshape(N, 1)
    zeros = jnp.zeros((N, D), _f32)

    x = _node_lin(node_attr, node_lin_W, node_lin_b.reshape(1, D))

    def conv(i, xin, res, residual):
        wm = conv_Wm[i]
        eb2 = _edge_pre(eap, esp, wm[2 * D:2 * D + DE], conv_slot[i])
        xs, xd2p = _pre_node(xin, batch2, wm[:D], wm[D:2 * D], u,
                             wm[2 * D + DE:], conv_bm[i].reshape(1, D))
        agg2 = _sc_edges(xs, xd2p, eb2, srcp, dstp, zeros)
        wu = conv_Wu[i]
        return _update(xin, agg2[0], agg2[1], res, wu[:D], wu[D:],
                       conv_bu[i].reshape(1, D), residual)

    ci = 0
    for _ in range(2):
        h = conv(ci, x, x, False)
        x = conv(ci + 1, h, x, True)
        ci += 2

    g, hv = _heads(x, gate_W1, gate_b1.reshape(1, 128), gate_W2,
                   gate_b2.reshape(1, 128), _pad_w3(gate_W3),
                   nn_W1, nn_b1.reshape(1, 128), nn_W2,
                   nn_b2.reshape(1, 128), _pad_w3(nn_W3))
    out = _pool(g, hv, batch2, gate_b3.reshape(1, 1), nn_b3.reshape(1, 1))
    return out.reshape(B, 1)
